# baseline (speedup 1.0000x reference)
import functools

import jax
import jax.numpy as jnp
from jax import lax
from jax.experimental import pallas as pl
from jax.experimental.pallas import tpu as pltpu
from jax.experimental.pallas import tpu_sc as plsc

B = 4096
L = 200
D = 64
NC = 2
NS = 16
NW = NC * NS
RPW = B // NW
C0 = 128
C1 = L - C0


def _pool_body(table, xw, xi, yw, yi, out_d, out_l,
               idx_v, w_v, rows_a, rows_b, out_v, sem_a, sem_b):
  wid = lax.axis_index("s") * NC + lax.axis_index("c")
  base = wid * RPW

  def start(r, buf, sem):
    pltpu.async_copy(table.at[idx_v.at[r, pl.ds(0, C0)]],
                     buf.at[pl.ds(0, C0)], sem)
    pltpu.async_copy(table.at[idx_v.at[r, pl.ds(C0, C1)]],
                     buf.at[pl.ds(C0, C1)], sem)

  def wait(r, buf, sem):
    pltpu.make_async_copy(table.at[idx_v.at[r, pl.ds(0, C0)]],
                          buf.at[pl.ds(0, C0)], sem).wait()
    pltpu.make_async_copy(table.at[idx_v.at[r, pl.ds(C0, C1)]],
                          buf.at[pl.ds(C0, C1)], sem).wait()

  def accum(r, buf):
    zero = jnp.zeros((16,), jnp.float32)
    himask = jnp.full((16,), 0xFFFF0000, jnp.uint32)

    def fma(l, wl, acc):
      a0, a1, a2, a3 = acc
      v0 = buf[l, pl.ds(0, 16)]
      v1 = buf[l, pl.ds(16, 16)]
      a0 = a0 + lax.bitcast_convert_type(v0 << 16, jnp.float32) * wl
      a1 = a1 + lax.bitcast_convert_type(v1 << 16, jnp.float32) * wl
      a2 = a2 + lax.bitcast_convert_type(v0 & himask, jnp.float32) * wl
      a3 = a3 + lax.bitcast_convert_type(v1 & himask, jnp.float32) * wl
      return (a0, a1, a2, a3)

    def group(lc, accs):
      accA, accB = accs
      wv = w_v[r, pl.ds(lc * 16, 16)]
      for ll in range(0, 16, 2):
        accA = fma(lc * 16 + ll, wv[ll], accA)
        accB = fma(lc * 16 + ll + 1, wv[ll + 1], accB)
      return (accA, accB)

    z4 = (zero, zero, zero, zero)
    accA, accB = lax.fori_loop(0, L // 16, group, (z4, z4))
    wv = w_v[r, pl.ds(L - 16, 16)]
    for ll in range(8, 16, 2):
      accA = fma(L - 16 + ll, wv[ll], accA)
      accB = fma(L - 16 + ll + 1, wv[ll + 1], accB)
    out_v[r, pl.ds(0, 16)] = accA[0] + accB[0]
    out_v[r, pl.ds(16, 16)] = accA[1] + accB[1]
    out_v[r, pl.ds(32, 16)] = accA[2] + accB[2]
    out_v[r, pl.ds(48, 16)] = accA[3] + accB[3]

  for wgt, ind, out in ((xw, xi, out_d), (yw, yi, out_l)):
    pltpu.sync_copy(ind.at[pl.ds(base, RPW)], idx_v)
    pltpu.sync_copy(wgt.at[pl.ds(base, RPW)], w_v)
    start(0, rows_a, sem_a)

    def step(rr):
      start(rr + 1, rows_b, sem_b)
      wait(rr, rows_a, sem_a)
      accum(rr, rows_a)

      @pl.when(rr + 2 < RPW)
      def _():
        start(rr + 2, rows_a, sem_a)

      wait(rr + 1, rows_b, sem_b)
      accum(rr + 1, rows_b)

    pl.loop(0, RPW, step=2)(step)
    pltpu.sync_copy(out_v, out.at[pl.ds(base, RPW)])


@functools.partial(jax.jit, donate_argnums=())
def _pool(table, xw, xi, yw, yi):
  mesh = plsc.VectorSubcoreMesh(core_axis_name="c", subcore_axis_name="s")
  k = pl.kernel(
      _pool_body,
      mesh=mesh,
      compiler_params=pltpu.CompilerParams(use_tc_tiling_on_sc=False),
      name="pool_sc",
      out_type=(jax.ShapeDtypeStruct((B, D), jnp.float32),
                jax.ShapeDtypeStruct((B, D), jnp.float32)),
      scratch_types=[
          pltpu.VMEM((RPW, L), jnp.int32),
          pltpu.VMEM((RPW, L), jnp.float32),
          pltpu.VMEM((L, D // 2), jnp.uint32),
          pltpu.VMEM((L, D // 2), jnp.uint32),
          pltpu.VMEM((RPW, D), jnp.float32),
          pltpu.SemaphoreType.DMA,
          pltpu.SemaphoreType.DMA,
      ],
  )
  return k(table, xw, xi, yw, yi)


VK = 1000000
VKP = 1000064
TW = 32768


def _tr_body(tin_ref, eye_ref, out_ref):
  blk = lax.dot_general(tin_ref[...], eye_ref[...], (((0,), (0,)), ((), ())),
                        preferred_element_type=jnp.float32)
  lo = lax.bitcast_convert_type(blk[:, 0:32].astype(jnp.bfloat16),
                                jnp.uint16).astype(jnp.uint32)
  hi = lax.bitcast_convert_type(blk[:, 32:64].astype(jnp.bfloat16),
                                jnp.uint16).astype(jnp.uint32)
  w = lo | (hi << 16)
  b4 = w.reshape(TW // 128, 128, D // 2)
  pk = jnp.concatenate([b4[:, 0:32, :], b4[:, 32:64, :],
                        b4[:, 64:96, :], b4[:, 96:128, :]], axis=2)
  out_ref[...] = pk.reshape(TW // 4, 128)


def _transpose(tableT, eye):
  return pl.pallas_call(
      _tr_body,
      grid=(pl.cdiv(VK, TW),),
      in_specs=[
          pl.BlockSpec((D, TW), lambda i: (0, i)),
          pl.BlockSpec((D, D), lambda i: (0, 0)),
      ],
      out_specs=pl.BlockSpec((TW // 4, 128), lambda i: (i, 0)),
      out_shape=jax.ShapeDtypeStruct((VKP // 4, 128), jnp.uint32),
  )(tableT, eye)


TILE = 256


def _sim_body(doc_ref, lbl_ref, w_ref, b_ref, out_ref, lrep_ref):
  i = pl.program_id(0)

  @pl.when(i == 0)
  def _():
    rep = jnp.maximum(
        jnp.dot(lbl_ref[...], w_ref[...],
                preferred_element_type=jnp.float32) + b_ref[...], 0.0)
    n = jnp.sqrt(jnp.sum(rep * rep, axis=1, keepdims=True))
    lrep_ref[...] = rep / jnp.maximum(n, 1e-12)

  drep = jnp.maximum(
      jnp.dot(doc_ref[...], w_ref[...],
              preferred_element_type=jnp.float32) + b_ref[...], 0.0)
  dn = jnp.sqrt(jnp.sum(drep * drep, axis=1, keepdims=True))
  drep = drep / jnp.maximum(dn, 1e-12)
  out_ref[...] = lax.dot_general(
      drep, lrep_ref[...], (((1,), (1,)), ((), ())),
      preferred_element_type=jnp.float32)


def _sim(pooled_doc, pooled_lbl, W, b2):
  return pl.pallas_call(
      _sim_body,
      grid=(B // TILE,),
      in_specs=[
          pl.BlockSpec((TILE, D), lambda i: (i, 0)),
          pl.BlockSpec((B, D), lambda i: (0, 0)),
          pl.BlockSpec((D, D), lambda i: (0, 0)),
          pl.BlockSpec((1, D), lambda i: (0, 0)),
      ],
      out_specs=pl.BlockSpec((TILE, B), lambda i: (i, 0)),
      out_shape=jax.ShapeDtypeStruct((B, B), jnp.float32),
      scratch_shapes=[pltpu.VMEM((B, D), jnp.float32)],
  )(pooled_doc, pooled_lbl, W, b2)


def kernel(X, X_ind, YX, YX_ind, table, W_fine, b_fine):
  tablep = _transpose(table.T, jnp.eye(D, dtype=jnp.float32))
  tablep = tablep.reshape(VKP, D // 2)
  def remap(i):
    return (i & ~jnp.int32(127)) + 4 * (i & 31) + ((i >> 5) & 3)
  pooled_doc, pooled_lbl = _pool(tablep, X, remap(X_ind), YX, remap(YX_ind))
  return _sim(pooled_doc, pooled_lbl, W_fine, b_fine.reshape(1, D))


# speedup vs baseline: 1.0154x; 1.0154x over previous
import functools

import jax
import jax.numpy as jnp
from jax import lax
from jax.experimental import pallas as pl
from jax.experimental.pallas import tpu as pltpu
from jax.experimental.pallas import tpu_sc as plsc

B = 4096
L = 200
D = 64
NC = 2
NS = 16
NW = NC * NS
RPW = B // NW
C0 = 128
C1 = L - C0


def _pool_body(table, xw, xi, yw, yi, out_d, out_l,
               idx_v, w_v, rows_a, rows_b, rows_c, rows_d, out_v,
               sem_a, sem_b, sem_c, sem_d):
  wid = lax.axis_index("s") * NC + lax.axis_index("c")
  base = wid * RPW

  def start(r, buf, sem):
    pltpu.async_copy(table.at[idx_v.at[r, pl.ds(0, C0)]],
                     buf.at[pl.ds(0, C0)], sem)
    pltpu.async_copy(table.at[idx_v.at[r, pl.ds(C0, C1)]],
                     buf.at[pl.ds(C0, C1)], sem)

  def wait(r, buf, sem):
    pltpu.make_async_copy(table.at[idx_v.at[r, pl.ds(0, C0)]],
                          buf.at[pl.ds(0, C0)], sem).wait()
    pltpu.make_async_copy(table.at[idx_v.at[r, pl.ds(C0, C1)]],
                          buf.at[pl.ds(C0, C1)], sem).wait()

  def accum(r, buf):
    zero = jnp.zeros((16,), jnp.float32)
    himask = jnp.full((16,), 0xFFFF0000, jnp.uint32)

    def fma(l, wl, acc):
      a0, a1, a2, a3 = acc
      v0 = buf[l, pl.ds(0, 16)]
      v1 = buf[l, pl.ds(16, 16)]
      a0 = a0 + lax.bitcast_convert_type(v0 << 16, jnp.float32) * wl
      a1 = a1 + lax.bitcast_convert_type(v1 << 16, jnp.float32) * wl
      a2 = a2 + lax.bitcast_convert_type(v0 & himask, jnp.float32) * wl
      a3 = a3 + lax.bitcast_convert_type(v1 & himask, jnp.float32) * wl
      return (a0, a1, a2, a3)

    def group(lc, accs):
      accA, accB = accs
      wv = w_v[r, pl.ds(lc * 16, 16)]
      for ll in range(0, 16, 2):
        accA = fma(lc * 16 + ll, wv[ll], accA)
        accB = fma(lc * 16 + ll + 1, wv[ll + 1], accB)
      return (accA, accB)

    z4 = (zero, zero, zero, zero)
    accA, accB = lax.fori_loop(0, L // 16, group, (z4, z4))
    wv = w_v[r, pl.ds(L - 16, 16)]
    for ll in range(8, 16, 2):
      accA = fma(L - 16 + ll, wv[ll], accA)
      accB = fma(L - 16 + ll + 1, wv[ll + 1], accB)
    out_v[r, pl.ds(0, 16)] = accA[0] + accB[0]
    out_v[r, pl.ds(16, 16)] = accA[1] + accB[1]
    out_v[r, pl.ds(32, 16)] = accA[2] + accB[2]
    out_v[r, pl.ds(48, 16)] = accA[3] + accB[3]

  bufs = (rows_a, rows_b, rows_c, rows_d)
  sems = (sem_a, sem_b, sem_c, sem_d)

  for wgt, ind, out in ((xw, xi, out_d), (yw, yi, out_l)):
    pltpu.sync_copy(ind.at[pl.ds(base, RPW)], idx_v)
    pltpu.sync_copy(wgt.at[pl.ds(base, RPW)], w_v)
    for p in range(3):
      start(p, bufs[p], sems[p])

    def step(rr):
      for p in range(4):
        nxt = rr + p + 3
        @pl.when(nxt < RPW)
        def _():
          start(nxt, bufs[(p + 3) % 4], sems[(p + 3) % 4])
        wait(rr + p, bufs[p], sems[p])
        accum(rr + p, bufs[p])

    pl.loop(0, RPW, step=4)(step)
    pltpu.sync_copy(out_v, out.at[pl.ds(base, RPW)])


@functools.partial(jax.jit, donate_argnums=())
def _pool(table, xw, xi, yw, yi):
  mesh = plsc.VectorSubcoreMesh(core_axis_name="c", subcore_axis_name="s")
  k = pl.kernel(
      _pool_body,
      mesh=mesh,
      compiler_params=pltpu.CompilerParams(use_tc_tiling_on_sc=False),
      name="pool_sc",
      out_type=(jax.ShapeDtypeStruct((B, D), jnp.float32),
                jax.ShapeDtypeStruct((B, D), jnp.float32)),
      scratch_types=[
          pltpu.VMEM((RPW, L), jnp.int32),
          pltpu.VMEM((RPW, L), jnp.float32),
          pltpu.VMEM((L, D // 2), jnp.uint32),
          pltpu.VMEM((L, D // 2), jnp.uint32),
          pltpu.VMEM((L, D // 2), jnp.uint32),
          pltpu.VMEM((L, D // 2), jnp.uint32),
          pltpu.VMEM((RPW, D), jnp.float32),
          pltpu.SemaphoreType.DMA,
          pltpu.SemaphoreType.DMA,
          pltpu.SemaphoreType.DMA,
          pltpu.SemaphoreType.DMA,
      ],
  )
  return k(table, xw, xi, yw, yi)


VK = 1000000
VKP = 1000064
TW = 32768


def _tr_body(tin_ref, eye_ref, out_ref):
  blk = lax.dot_general(tin_ref[...], eye_ref[...], (((0,), (0,)), ((), ())),
                        preferred_element_type=jnp.float32)
  lo = lax.bitcast_convert_type(blk[:, 0:32].astype(jnp.bfloat16),
                                jnp.uint16).astype(jnp.uint32)
  hi = lax.bitcast_convert_type(blk[:, 32:64].astype(jnp.bfloat16),
                                jnp.uint16).astype(jnp.uint32)
  w = lo | (hi << 16)
  b4 = w.reshape(TW // 128, 128, D // 2)
  pk = jnp.concatenate([b4[:, 0:32, :], b4[:, 32:64, :],
                        b4[:, 64:96, :], b4[:, 96:128, :]], axis=2)
  out_ref[...] = pk.reshape(TW // 4, 128)


def _transpose(tableT, eye):
  return pl.pallas_call(
      _tr_body,
      grid=(pl.cdiv(VK, TW),),
      in_specs=[
          pl.BlockSpec((D, TW), lambda i: (0, i)),
          pl.BlockSpec((D, D), lambda i: (0, 0)),
      ],
      out_specs=pl.BlockSpec((TW // 4, 128), lambda i: (i, 0)),
      out_shape=jax.ShapeDtypeStruct((VKP // 4, 128), jnp.uint32),
  )(tableT, eye)


TILE = 256


def _sim_body(doc_ref, lbl_ref, w_ref, b_ref, out_ref, lrep_ref):
  i = pl.program_id(0)

  @pl.when(i == 0)
  def _():
    rep = jnp.maximum(
        jnp.dot(lbl_ref[...], w_ref[...],
                preferred_element_type=jnp.float32) + b_ref[...], 0.0)
    n = jnp.sqrt(jnp.sum(rep * rep, axis=1, keepdims=True))
    lrep_ref[...] = rep / jnp.maximum(n, 1e-12)

  drep = jnp.maximum(
      jnp.dot(doc_ref[...], w_ref[...],
              preferred_element_type=jnp.float32) + b_ref[...], 0.0)
  dn = jnp.sqrt(jnp.sum(drep * drep, axis=1, keepdims=True))
  drep = drep / jnp.maximum(dn, 1e-12)
  out_ref[...] = lax.dot_general(
      drep, lrep_ref[...], (((1,), (1,)), ((), ())),
      preferred_element_type=jnp.float32)


def _sim(pooled_doc, pooled_lbl, W, b2):
  return pl.pallas_call(
      _sim_body,
      grid=(B // TILE,),
      in_specs=[
          pl.BlockSpec((TILE, D), lambda i: (i, 0)),
          pl.BlockSpec((B, D), lambda i: (0, 0)),
          pl.BlockSpec((D, D), lambda i: (0, 0)),
          pl.BlockSpec((1, D), lambda i: (0, 0)),
      ],
      out_specs=pl.BlockSpec((TILE, B), lambda i: (i, 0)),
      out_shape=jax.ShapeDtypeStruct((B, B), jnp.float32),
      scratch_shapes=[pltpu.VMEM((B, D), jnp.float32)],
  )(pooled_doc, pooled_lbl, W, b2)


def kernel(X, X_ind, YX, YX_ind, table, W_fine, b_fine):
  tablep = _transpose(table.T, jnp.eye(D, dtype=jnp.float32))
  tablep = tablep.reshape(VKP, D // 2)
  def remap(i):
    return (i & ~jnp.int32(127)) + 4 * (i & 31) + ((i >> 5) & 3)
  pooled_doc, pooled_lbl = _pool(tablep, X, remap(X_ind), YX, remap(YX_ind))
  return _sim(pooled_doc, pooled_lbl, W_fine, b_fine.reshape(1, D))


# speedup vs baseline: 1.1853x; 1.1673x over previous
import functools

import jax
import jax.numpy as jnp
from jax import lax
from jax.experimental import pallas as pl
from jax.experimental.pallas import tpu as pltpu
from jax.experimental.pallas import tpu_sc as plsc

B = 4096
L = 200
D = 64
NC = 2
NS = 16
NW = NC * NS
RPW = B // NW
C0 = 128
C1 = L - C0


def _pool_body(table, xw, xi, yw, yi, out_d, out_l,
               idx_v, w_v, rows_a, rows_b, out_v, sem_a, sem_b):
  wid = lax.axis_index("s") * NC + lax.axis_index("c")
  base = wid * RPW

  def start(r, buf, sem):
    pltpu.async_copy(table.at[idx_v.at[r, pl.ds(0, C0)]],
                     buf.at[pl.ds(0, C0)], sem)
    pltpu.async_copy(table.at[idx_v.at[r, pl.ds(C0, C1)]],
                     buf.at[pl.ds(C0, C1)], sem)

  def wait(r, buf, sem):
    pltpu.make_async_copy(table.at[idx_v.at[r, pl.ds(0, C0)]],
                          buf.at[pl.ds(0, C0)], sem).wait()
    pltpu.make_async_copy(table.at[idx_v.at[r, pl.ds(C0, C1)]],
                          buf.at[pl.ds(C0, C1)], sem).wait()

  def accum(r, buf):
    zero = jnp.zeros((16,), jnp.float32)

    def fma(l, wl, acc):
      a0, a1, a2, a3 = acc
      a0 = a0 + buf[l, pl.ds(0, 16)] * wl
      a1 = a1 + buf[l, pl.ds(16, 16)] * wl
      a2 = a2 + buf[l, pl.ds(32, 16)] * wl
      a3 = a3 + buf[l, pl.ds(48, 16)] * wl
      return (a0, a1, a2, a3)

    def group(lc, acc):
      wv = w_v[r, pl.ds(lc * 16, 16)]
      for ll in range(16):
        acc = fma(lc * 16 + ll, wv[ll], acc)
      return acc

    acc = lax.fori_loop(0, L // 16, group, (zero, zero, zero, zero))
    wv = w_v[r, pl.ds(L - 16, 16)]
    for ll in range(8, 16):
      acc = fma(L - 16 + ll, wv[ll], acc)
    a0, a1, a2, a3 = acc
    out_v[r, pl.ds(0, 16)] = a0
    out_v[r, pl.ds(16, 16)] = a1
    out_v[r, pl.ds(32, 16)] = a2
    out_v[r, pl.ds(48, 16)] = a3

  for wgt, ind, out in ((xw, xi, out_d), (yw, yi, out_l)):
    pltpu.sync_copy(ind.at[pl.ds(base, RPW)], idx_v)
    pltpu.sync_copy(wgt.at[pl.ds(base, RPW)], w_v)
    start(0, rows_a, sem_a)

    def step(rr):
      start(rr + 1, rows_b, sem_b)
      wait(rr, rows_a, sem_a)
      accum(rr, rows_a)

      @pl.when(rr + 2 < RPW)
      def _():
        start(rr + 2, rows_a, sem_a)

      wait(rr + 1, rows_b, sem_b)
      accum(rr + 1, rows_b)

    pl.loop(0, RPW, step=2)(step)
    pltpu.sync_copy(out_v, out.at[pl.ds(base, RPW)])


@functools.partial(jax.jit, donate_argnums=())
def _pool(table, xw, xi, yw, yi):
  mesh = plsc.VectorSubcoreMesh(core_axis_name="c", subcore_axis_name="s")
  k = pl.kernel(
      _pool_body,
      mesh=mesh,
      compiler_params=pltpu.CompilerParams(use_tc_tiling_on_sc=False),
      name="pool_sc",
      out_type=(jax.ShapeDtypeStruct((B, D), jnp.float32),
                jax.ShapeDtypeStruct((B, D), jnp.float32)),
      scratch_types=[
          pltpu.VMEM((RPW, L), jnp.int32),
          pltpu.VMEM((RPW, L), jnp.float32),
          pltpu.VMEM((L, D), jnp.float32),
          pltpu.VMEM((L, D), jnp.float32),
          pltpu.VMEM((RPW, D), jnp.float32),
          pltpu.SemaphoreType.DMA,
          pltpu.SemaphoreType.DMA,
      ],
  )
  return k(table, xw, xi, yw, yi)


VK = 1000000
VKP = 1000064
TW = 32768


def _tr_body(tin_ref, eye_ref, out_ref):
  blk = lax.dot_general(tin_ref[...], eye_ref[...], (((0,), (0,)), ((), ())),
                        preferred_element_type=jnp.float32)
  b4 = blk.reshape(TW // 128, 128, D)
  pk = jnp.concatenate([b4[:, 0:64, :], b4[:, 64:128, :]], axis=2)
  out_ref[...] = pk.reshape(TW // 2, 128)


def _transpose(tableT, eye):
  return pl.pallas_call(
      _tr_body,
      grid=(pl.cdiv(VK, TW),),
      in_specs=[
          pl.BlockSpec((D, TW), lambda i: (0, i)),
          pl.BlockSpec((D, D), lambda i: (0, 0)),
      ],
      out_specs=pl.BlockSpec((TW // 2, 128), lambda i: (i, 0)),
      out_shape=jax.ShapeDtypeStruct((VKP // 2, 128), jnp.float32),
  )(tableT, eye)


TILE = 256


def _sim_body(doc_ref, lbl_ref, w_ref, b_ref, out_ref, lrep_ref):
  i = pl.program_id(0)

  @pl.when(i == 0)
  def _():
    rep = jnp.maximum(
        jnp.dot(lbl_ref[...], w_ref[...],
                preferred_element_type=jnp.float32) + b_ref[...], 0.0)
    n = jnp.sqrt(jnp.sum(rep * rep, axis=1, keepdims=True))
    lrep_ref[...] = rep / jnp.maximum(n, 1e-12)

  drep = jnp.maximum(
      jnp.dot(doc_ref[...], w_ref[...],
              preferred_element_type=jnp.float32) + b_ref[...], 0.0)
  dn = jnp.sqrt(jnp.sum(drep * drep, axis=1, keepdims=True))
  drep = drep / jnp.maximum(dn, 1e-12)
  out_ref[...] = lax.dot_general(
      drep, lrep_ref[...], (((1,), (1,)), ((), ())),
      preferred_element_type=jnp.float32)


def _sim(pooled_doc, pooled_lbl, W, b2):
  return pl.pallas_call(
      _sim_body,
      grid=(B // TILE,),
      in_specs=[
          pl.BlockSpec((TILE, D), lambda i: (i, 0)),
          pl.BlockSpec((B, D), lambda i: (0, 0)),
          pl.BlockSpec((D, D), lambda i: (0, 0)),
          pl.BlockSpec((1, D), lambda i: (0, 0)),
      ],
      out_specs=pl.BlockSpec((TILE, B), lambda i: (i, 0)),
      out_shape=jax.ShapeDtypeStruct((B, B), jnp.float32),
      scratch_shapes=[pltpu.VMEM((B, D), jnp.float32)],
  )(pooled_doc, pooled_lbl, W, b2)


def kernel(X, X_ind, YX, YX_ind, table, W_fine, b_fine):
  tablep = _transpose(table.T, jnp.eye(D, dtype=jnp.float32))
  tablep = tablep.reshape(VKP, D)
  def remap(i):
    return (i & ~jnp.int32(127)) + 2 * (i & 63) + ((i >> 6) & 1)
  pooled_doc, pooled_lbl = _pool(tablep, X, remap(X_ind), YX, remap(YX_ind))
  return _sim(pooled_doc, pooled_lbl, W_fine, b_fine.reshape(1, D))
